# Initial kernel scaffold; baseline (speedup 1.0000x reference)
import jax
import jax.numpy as jnp
from jax.experimental import pallas as pl


def kernel(x, G_indices, G_values, W, b):
    raise NotImplementedError("write your pallas kernel here")


# speedup vs baseline: 3.9863x; 3.9863x over previous
import functools

import jax
import jax.numpy as jnp
from jax import lax
from jax.experimental import pallas as pl
from jax.experimental.pallas import tpu as pltpu
from jax.experimental.pallas import tpu_sc as plsc

N = 10000
E = 320000
D = 128

NC = 2
NS = 16
NW = NC * NS
EPW = E // NW
B = 80
NCHUNK = EPW // B
RPA = 632
RPL = N - 15 * RPA


def _matmul_body(x_ref, w_ref, b_ref, o_ref):
    o_ref[...] = (
        jnp.dot(x_ref[...], w_ref[...], preferred_element_type=jnp.float32)
        + b_ref[...]
    )


def _xw(x, W, b):
    blk = 1000
    return pl.pallas_call(
        _matmul_body,
        grid=(N // blk,),
        in_specs=[
            pl.BlockSpec((blk, D), lambda i: (i, 0)),
            pl.BlockSpec((D, D), lambda i: (0, 0)),
            pl.BlockSpec((1, D), lambda i: (0, 0)),
        ],
        out_specs=pl.BlockSpec((blk, D), lambda i: (i, 0)),
        out_shape=jax.ShapeDtypeStruct((N, D), jnp.float32),
    )(x, W, b.reshape(1, D))


def _spmm_sc(xw, rows, cols, vals):
    mesh = plsc.VectorSubcoreMesh(core_axis_name="c", subcore_axis_name="s")

    @functools.partial(
        pl.kernel,
        out_type=jax.ShapeDtypeStruct((NC, N, D), jnp.float32),
        mesh=mesh,
        scratch_types=[
            pltpu.VMEM((B,), jnp.int32),
            pltpu.VMEM((B,), jnp.int32),
            pltpu.VMEM((B + 16,), jnp.float32),
            pltpu.VMEM((B, D), jnp.float32),
            pltpu.VMEM((8, D), jnp.float32),
            pltpu.VMEM_SHARED((N, D), jnp.float32),
            pltpu.SemaphoreType.DMA,
        ],
    )
    def k(xw_hbm, rows_hbm, cols_hbm, vals_hbm, out_hbm,
          colv, rowv, valv, gbuf, zbuf, acc, sem):
        c = lax.axis_index("c")
        s = lax.axis_index("s")
        wid = s * NC + c

        zero16 = jnp.zeros((16,), jnp.float32)
        for j in range(8):
            for t in range(D // 16):
                zbuf[j, pl.ds(t * 16, 16)] = zero16

        nz = jnp.where(s == NS - 1, RPL // 8, RPA // 8)

        def zinit(j, carry):
            pltpu.sync_copy(zbuf, acc.at[pl.ds(s * RPA + j * 8, 8)])
            return carry

        lax.fori_loop(0, nz, zinit, 0)
        plsc.subcore_barrier()

        def body(i, carry):
            base = wid * EPW + i * B
            pltpu.sync_copy(cols_hbm.at[pl.ds(base, B)], colv)
            pltpu.sync_copy(rows_hbm.at[pl.ds(base, B)], rowv)
            pltpu.sync_copy(vals_hbm.at[pl.ds(base, B)], valv.at[pl.ds(0, B)])
            pltpu.async_copy(xw_hbm.at[colv], gbuf, sem).wait()

            def scale(e, cc):
                v = valv[pl.ds(e, 16)][0]
                for t in range(D // 16):
                    gbuf[e, pl.ds(t * 16, 16)] = gbuf[e, pl.ds(t * 16, 16)] * v
                return cc

            lax.fori_loop(0, B, scale, 0)
            pltpu.sync_copy(gbuf, acc.at[rowv], add=True)
            return carry

        lax.fori_loop(0, NCHUNK, body, 0)

        plsc.subcore_barrier()

        @pl.when(s < NS - 1)
        def _wb_main():
            pltpu.sync_copy(acc.at[pl.ds(s * RPA, RPA)],
                            out_hbm.at[c, pl.ds(s * RPA, RPA)])

        @pl.when(s == NS - 1)
        def _wb_last():
            pltpu.sync_copy(acc.at[pl.ds((NS - 1) * RPA, RPL)],
                            out_hbm.at[c, pl.ds((NS - 1) * RPA, RPL)])

    return k(xw, rows, cols, vals)


def _add_body(p_ref, o_ref):
    o_ref[...] = p_ref[0] + p_ref[1]


def _combine(partials):
    blk = 1000
    return pl.pallas_call(
        _add_body,
        grid=(N // blk,),
        in_specs=[pl.BlockSpec((NC, blk, D), lambda i: (0, i, 0))],
        out_specs=pl.BlockSpec((blk, D), lambda i: (i, 0)),
        out_shape=jax.ShapeDtypeStruct((N, D), jnp.float32),
    )(partials)


def kernel(x, G_indices, G_values, W, b):
    xw = _xw(x, W, b)
    partials = _spmm_sc(xw, G_indices[0], G_indices[1], G_values)
    return _combine(partials)


# speedup vs baseline: 8.9514x; 2.2455x over previous
import functools

import jax
import jax.numpy as jnp
from jax import lax
from jax.experimental import pallas as pl
from jax.experimental.pallas import tpu as pltpu
from jax.experimental.pallas import tpu_sc as plsc

N = 10000
E = 320000
D = 128

NC = 2
NS = 16
NW = NC * NS
EPW = E // NW
B = 80
NCHUNK = EPW // B
RPA = 632
RPL = N - 15 * RPA
ZR = 104


def _matmul_body(x_ref, w_ref, b_ref, o_ref):
    o_ref[...] = (
        jnp.dot(x_ref[...], w_ref[...], preferred_element_type=jnp.float32)
        + b_ref[...]
    )


def _xw(x, W, b):
    blk = 1000
    return pl.pallas_call(
        _matmul_body,
        grid=(N // blk,),
        in_specs=[
            pl.BlockSpec((blk, D), lambda i: (i, 0)),
            pl.BlockSpec((D, D), lambda i: (0, 0)),
            pl.BlockSpec((1, D), lambda i: (0, 0)),
        ],
        out_specs=pl.BlockSpec((blk, D), lambda i: (i, 0)),
        out_shape=jax.ShapeDtypeStruct((N, D), jnp.float32),
    )(x, W, b.reshape(1, D))


def _spmm_sc(xw, rows, cols, vals):
    mesh = plsc.VectorSubcoreMesh(core_axis_name="c", subcore_axis_name="s")

    @functools.partial(
        pl.kernel,
        out_type=jax.ShapeDtypeStruct((NC, N, D), jnp.float32),
        mesh=mesh,
        scratch_types=[
            [pltpu.VMEM((B,), jnp.int32)] * 3,
            [pltpu.VMEM((B,), jnp.int32)] * 3,
            [pltpu.VMEM((B,), jnp.float32)] * 3,
            [pltpu.VMEM((B, D), jnp.float32)] * 3,
            pltpu.VMEM((ZR, D), jnp.float32),
            pltpu.VMEM_SHARED((N, D), jnp.float32),
            [pltpu.SemaphoreType.DMA] * 3,
            [pltpu.SemaphoreType.DMA] * 3,
        ],
    )
    def k(xw_hbm, rows_hbm, cols_hbm, vals_hbm, out_hbm,
          rv, cv, vv, gb, zbuf, acc, semi, semg):
        c = lax.axis_index("c")
        s = lax.axis_index("s")
        wid = s * NC + c
        ebase = wid * EPW

        def fetch_idx(i, p):
            pltpu.async_copy(
                rows_hbm.at[pl.ds(ebase + i * B, B)], rv[p], semi[p])
            pltpu.async_copy(
                cols_hbm.at[pl.ds(ebase + i * B, B)], cv[p], semi[p])
            pltpu.async_copy(
                vals_hbm.at[pl.ds(ebase + i * B, B)], vv[p], semi[p])

        def wait_idx(i, p):
            pltpu.make_async_copy(
                rows_hbm.at[pl.ds(ebase + i * B, B)], rv[p], semi[p]).wait()
            pltpu.make_async_copy(
                cols_hbm.at[pl.ds(ebase + i * B, B)], cv[p], semi[p]).wait()
            pltpu.make_async_copy(
                vals_hbm.at[pl.ds(ebase + i * B, B)], vv[p], semi[p]).wait()

        def gather(p):
            pltpu.async_copy(xw_hbm.at[cv[p]], gb[p], semg[p])

        def wait_gather(p):
            pltpu.make_async_copy(xw_hbm.at[cv[p]], gb[p], semg[p]).wait()

        fetch_idx(0, 0)
        fetch_idx(1, 1)
        fetch_idx(2, 2)

        zero16 = jnp.zeros((16,), jnp.float32)

        def zfill(j, cc):
            for t in range(D // 16):
                zbuf[j, pl.ds(t * 16, 16)] = zero16
            return cc

        lax.fori_loop(0, ZR, zfill, 0)
        for kk in range(5):
            pltpu.sync_copy(zbuf, acc.at[pl.ds(s * RPA + kk * ZR, ZR)])

        @pl.when(s < NS - 1)
        def _zrest():
            pltpu.sync_copy(zbuf, acc.at[pl.ds(s * RPA + 5 * ZR, ZR)])
            pltpu.sync_copy(zbuf.at[pl.ds(0, 8)],
                            acc.at[pl.ds(s * RPA + 6 * ZR, 8)])

        plsc.subcore_barrier()

        wait_idx(0, 0)
        gather(0)
        wait_idx(1, 1)
        gather(1)

        def process(p):
            def sg(g, cc):
                vvec = vv[p][pl.ds(g * 16, 16)]
                for e in range(16):
                    v = vvec[e]
                    r = g * 16 + e
                    for t in range(D // 16):
                        gb[p][r, pl.ds(t * 16, 16)] = (
                            gb[p][r, pl.ds(t * 16, 16)] * v)
                return cc

            lax.fori_loop(0, B // 16, sg, 0)
            pltpu.sync_copy(gb[p], acc.at[rv[p]], add=True)

        def step(i, p):
            @pl.when(i + 2 <= NCHUNK - 1)
            def _start_next_gather():
                wait_idx(i + 2, (p + 2) % 3)
                gather((p + 2) % 3)

            wait_gather(p)
            process(p)

            @pl.when(i + 3 <= NCHUNK - 1)
            def _prefetch_idx():
                fetch_idx(i + 3, p)

        def body(j, cc):
            step(3 * j, 0)
            step(3 * j + 1, 1)
            step(3 * j + 2, 2)
            return cc

        lax.fori_loop(0, NCHUNK // 3, body, 0)
        step(jnp.int32(NCHUNK - 2), 0)
        step(jnp.int32(NCHUNK - 1), 1)

        plsc.subcore_barrier()

        @pl.when(s < NS - 1)
        def _wb_main():
            pltpu.sync_copy(acc.at[pl.ds(s * RPA, RPA)],
                            out_hbm.at[c, pl.ds(s * RPA, RPA)])

        @pl.when(s == NS - 1)
        def _wb_last():
            pltpu.sync_copy(acc.at[pl.ds((NS - 1) * RPA, RPL)],
                            out_hbm.at[c, pl.ds((NS - 1) * RPA, RPL)])

    return k(xw, rows, cols, vals)


def _add_body(p_ref, o_ref):
    o_ref[...] = p_ref[0] + p_ref[1]


def _combine(partials):
    blk = 1000
    return pl.pallas_call(
        _add_body,
        grid=(N // blk,),
        in_specs=[pl.BlockSpec((NC, blk, D), lambda i: (0, i, 0))],
        out_specs=pl.BlockSpec((blk, D), lambda i: (i, 0)),
        out_shape=jax.ShapeDtypeStruct((N, D), jnp.float32),
    )(partials)


def kernel(x, G_indices, G_values, W, b):
    xw = _xw(x, W, b)
    partials = _spmm_sc(xw, G_indices[0], G_indices[1], G_values)
    return _combine(partials)


# speedup vs baseline: 12.0480x; 1.3459x over previous
import functools

import jax
import jax.numpy as jnp
from jax import lax
from jax.experimental import pallas as pl
from jax.experimental.pallas import tpu as pltpu
from jax.experimental.pallas import tpu_sc as plsc

N = 10000
E = 320000
D = 128

NC = 2
NS = 16
NW = NC * NS
EPW = E // NW
B = 80
NCHUNK = EPW // B
KG = 4
KI = 5
SUP = 20
RPA = 632
RPL = N - 15 * RPA
ZR = 32


def _matmul_body(x_ref, w_ref, b_ref, o_ref):
    o_ref[...] = (
        jnp.dot(x_ref[...], w_ref[...], preferred_element_type=jnp.float32)
        + b_ref[...]
    )


def _xw(x, W, b):
    blk = 1000
    return pl.pallas_call(
        _matmul_body,
        grid=(N // blk,),
        in_specs=[
            pl.BlockSpec((blk, D), lambda i: (i, 0)),
            pl.BlockSpec((D, D), lambda i: (0, 0)),
            pl.BlockSpec((1, D), lambda i: (0, 0)),
        ],
        out_specs=pl.BlockSpec((blk, D), lambda i: (i, 0)),
        out_shape=jax.ShapeDtypeStruct((N, D), jnp.float32),
    )(x, W, b.reshape(1, D))


def _spmm_sc(xw, rows, cols, vals):
    mesh = plsc.VectorSubcoreMesh(core_axis_name="c", subcore_axis_name="s")

    @functools.partial(
        pl.kernel,
        out_type=jax.ShapeDtypeStruct((NC, N, D), jnp.float32),
        mesh=mesh,
        scratch_types=[
            [pltpu.VMEM((B,), jnp.int32)] * KI,
            [pltpu.VMEM((B,), jnp.int32)] * KI,
            [pltpu.VMEM((B + 16,), jnp.float32)] * KI,
            [pltpu.VMEM((B, D), jnp.float32)] * KG,
            pltpu.VMEM((ZR, D), jnp.float32),
            pltpu.VMEM_SHARED((N, D), jnp.float32),
            [pltpu.SemaphoreType.DMA] * KI,
            [pltpu.SemaphoreType.DMA] * KG,
            [pltpu.SemaphoreType.DMA] * KG,
        ],
    )
    def k(xw_hbm, rows_hbm, cols_hbm, vals_hbm, out_hbm,
          rv, cv, vv, gb, zbuf, acc, semi, semg, sems):
        c = lax.axis_index("c")
        s = lax.axis_index("s")
        wid = s * NC + c
        ebase = wid * EPW

        def fetch_idx(i, x):
            pltpu.async_copy(
                rows_hbm.at[pl.ds(ebase + i * B, B)], rv[x], semi[x])
            pltpu.async_copy(
                cols_hbm.at[pl.ds(ebase + i * B, B)], cv[x], semi[x])
            pltpu.async_copy(
                vals_hbm.at[pl.ds(ebase + i * B, B)],
                vv[x].at[pl.ds(0, B)], semi[x])

        def wait_idx(i, x):
            pltpu.make_async_copy(
                rows_hbm.at[pl.ds(ebase + i * B, B)], rv[x], semi[x]).wait()
            pltpu.make_async_copy(
                cols_hbm.at[pl.ds(ebase + i * B, B)], cv[x], semi[x]).wait()
            pltpu.make_async_copy(
                vals_hbm.at[pl.ds(ebase + i * B, B)],
                vv[x].at[pl.ds(0, B)], semi[x]).wait()

        def gather(x, g):
            pltpu.async_copy(xw_hbm.at[cv[x]], gb[g], semg[g])

        def wait_gather(x, g):
            pltpu.make_async_copy(xw_hbm.at[cv[x]], gb[g], semg[g]).wait()

        def scatter(x, g):
            pltpu.async_copy(gb[g], acc.at[rv[x]], sems[g], add=True)

        def wait_scatter(x, g):
            pltpu.make_async_copy(gb[g], acc.at[rv[x]], sems[g]).wait()

        fetch_idx(0, 0)
        fetch_idx(1, 1)
        fetch_idx(2, 2)

        zero16 = jnp.zeros((16,), jnp.float32)

        def zfill(j, cc):
            for t in range(D // 16):
                zbuf[j, pl.ds(t * 16, 16)] = zero16
            return cc

        lax.fori_loop(0, ZR, zfill, 0)

        def zinit(j, cc):
            pltpu.sync_copy(zbuf, acc.at[pl.ds(s * RPA + j * ZR, ZR)])
            return cc

        nz = jnp.where(s == NS - 1, 16, 19)
        lax.fori_loop(0, nz, zinit, 0)

        @pl.when(s < NS - 1)
        def _ztail():
            pltpu.sync_copy(zbuf.at[pl.ds(0, 24)],
                            acc.at[pl.ds(s * RPA + 19 * ZR, 24)])

        @pl.when(s == NS - 1)
        def _ztail_last():
            pltpu.sync_copy(zbuf.at[pl.ds(0, 8)],
                            acc.at[pl.ds(s * RPA + 16 * ZR, 8)])

        plsc.subcore_barrier()

        wait_idx(0, 0)
        gather(0, 0)
        wait_idx(1, 1)
        gather(1, 1)

        def scale(x, g):
            def sg(gi, cc):
                vvec = vv[x][pl.ds(gi * 4, 16)]
                for e in range(4):
                    v = vvec[e]
                    r = gi * 4 + e
                    for t in range(D // 16):
                        gb[g][r, pl.ds(t * 16, 16)] = (
                            gb[g][r, pl.ds(t * 16, 16)] * v)
                return cc

            lax.fori_loop(0, B // 4, sg, 0)

        def step(i, g, x):
            gp2 = (g + 2) % KG
            xp2 = (x + 2) % KI
            xp3 = (x + 3) % KI

            @pl.when(i >= 2)
            def _drain_scatter():
                wait_scatter(xp3, gp2)

            @pl.when(i + 3 <= NCHUNK - 1)
            def _prefetch_idx():
                fetch_idx(i + 3, xp3)

            @pl.when(i + 2 <= NCHUNK - 1)
            def _start_gather():
                wait_idx(i + 2, xp2)
                gather(xp2, gp2)

            wait_gather(x, g)
            scale(x, g)
            scatter(x, g)

        def body(ss, cc):
            for kk in range(SUP):
                step(ss * SUP + kk, kk % KG, kk % KI)
            return cc

        lax.fori_loop(0, NCHUNK // SUP, body, 0)
        for i in range(SUP * (NCHUNK // SUP), NCHUNK):
            step(jnp.int32(i), i % KG, i % KI)

        wait_scatter((NCHUNK - 2) % KI, (NCHUNK - 2) % KG)
        wait_scatter((NCHUNK - 1) % KI, (NCHUNK - 1) % KG)

        plsc.subcore_barrier()

        @pl.when(s < NS - 1)
        def _wb_main():
            pltpu.sync_copy(acc.at[pl.ds(s * RPA, RPA)],
                            out_hbm.at[c, pl.ds(s * RPA, RPA)])

        @pl.when(s == NS - 1)
        def _wb_last():
            pltpu.sync_copy(acc.at[pl.ds((NS - 1) * RPA, RPL)],
                            out_hbm.at[c, pl.ds((NS - 1) * RPA, RPL)])

    return k(xw, rows, cols, vals)


def _add_body(p_ref, o_ref):
    o_ref[...] = p_ref[0] + p_ref[1]


def _combine(partials):
    blk = 1000
    return pl.pallas_call(
        _add_body,
        grid=(N // blk,),
        in_specs=[pl.BlockSpec((NC, blk, D), lambda i: (0, i, 0))],
        out_specs=pl.BlockSpec((blk, D), lambda i: (i, 0)),
        out_shape=jax.ShapeDtypeStruct((N, D), jnp.float32),
    )(partials)


def kernel(x, G_indices, G_values, W, b):
    xw = _xw(x, W, b)
    partials = _spmm_sc(xw, G_indices[0], G_indices[1], G_values)
    return _combine(partials)


# speedup vs baseline: 12.0617x; 1.0011x over previous
import functools

import jax
import jax.numpy as jnp
from jax import lax
from jax.experimental import pallas as pl
from jax.experimental.pallas import tpu as pltpu
from jax.experimental.pallas import tpu_sc as plsc

N = 10000
E = 320000
D = 128

NC = 2
NS = 16
NW = NC * NS
EPW = E // NW
B = 80
NCHUNK = EPW // B
KG = 4
KI = 5
SUP = 20
RPA = 632
RPL = N - 15 * RPA
ZR = 32


def _matmul_body(x_ref, w_ref, b_ref, o_ref):
    o_ref[...] = (
        jnp.dot(x_ref[...], w_ref[...], preferred_element_type=jnp.float32)
        + b_ref[...]
    )


def _xw(x, W, b):
    blk = 1000
    return pl.pallas_call(
        _matmul_body,
        grid=(N // blk,),
        in_specs=[
            pl.BlockSpec((blk, D), lambda i: (i, 0)),
            pl.BlockSpec((D, D), lambda i: (0, 0)),
            pl.BlockSpec((1, D), lambda i: (0, 0)),
        ],
        out_specs=pl.BlockSpec((blk, D), lambda i: (i, 0)),
        out_shape=jax.ShapeDtypeStruct((N, D), jnp.float32),
    )(x, W, b.reshape(1, D))


def _spmm_sc(xw, rows, cols, vals):
    mesh = plsc.VectorSubcoreMesh(core_axis_name="c", subcore_axis_name="s")

    @functools.partial(
        pl.kernel,
        out_type=jax.ShapeDtypeStruct((NC, N, D), jnp.float32),
        mesh=mesh,
        scratch_types=[
            [pltpu.VMEM((B,), jnp.int32)] * KI,
            [pltpu.VMEM((B,), jnp.int32)] * KI,
            [pltpu.VMEM((B + 16,), jnp.float32)] * KI,
            [pltpu.VMEM((B, D), jnp.float32)] * KG,
            pltpu.VMEM((ZR, D), jnp.float32),
            pltpu.VMEM_SHARED((N, D), jnp.float32),
            [pltpu.SemaphoreType.DMA] * KI,
            [pltpu.SemaphoreType.DMA] * KG,
            [pltpu.SemaphoreType.DMA] * KG,
        ],
    )
    def k(xw_hbm, rows_hbm, cols_hbm, vals_hbm, out_hbm,
          rv, cv, vv, gb, zbuf, acc, semi, semg, sems):
        c = lax.axis_index("c")
        s = lax.axis_index("s")
        wid = s * NC + c
        ebase = wid * EPW

        def fetch_idx(i, x):
            pltpu.async_copy(
                rows_hbm.at[pl.ds(ebase + i * B, B)], rv[x], semi[x])
            pltpu.async_copy(
                cols_hbm.at[pl.ds(ebase + i * B, B)], cv[x], semi[x])
            pltpu.async_copy(
                vals_hbm.at[pl.ds(ebase + i * B, B)],
                vv[x].at[pl.ds(0, B)], semi[x])

        def wait_idx(i, x):
            pltpu.make_async_copy(
                rows_hbm.at[pl.ds(ebase + i * B, B)], rv[x], semi[x]).wait()
            pltpu.make_async_copy(
                cols_hbm.at[pl.ds(ebase + i * B, B)], cv[x], semi[x]).wait()
            pltpu.make_async_copy(
                vals_hbm.at[pl.ds(ebase + i * B, B)],
                vv[x].at[pl.ds(0, B)], semi[x]).wait()

        def gather(x, g):
            pltpu.async_copy(xw_hbm.at[cv[x]], gb[g], semg[g])

        def wait_gather(x, g):
            pltpu.make_async_copy(xw_hbm.at[cv[x]], gb[g], semg[g]).wait()

        def scatter(x, g):
            pltpu.async_copy(gb[g], acc.at[rv[x]], sems[g], add=True)

        def wait_scatter(x, g):
            pltpu.make_async_copy(gb[g], acc.at[rv[x]], sems[g]).wait()

        fetch_idx(0, 0)
        fetch_idx(1, 1)
        fetch_idx(2, 2)

        zero16 = jnp.zeros((16,), jnp.float32)

        def zfill(j, cc):
            for t in range(D // 16):
                zbuf[j, pl.ds(t * 16, 16)] = zero16
            return cc

        lax.fori_loop(0, ZR, zfill, 0)

        def zinit(j, cc):
            pltpu.sync_copy(zbuf, acc.at[pl.ds(s * RPA + j * ZR, ZR)])
            return cc

        nz = jnp.where(s == NS - 1, 16, 19)
        lax.fori_loop(0, nz, zinit, 0)

        @pl.when(s < NS - 1)
        def _ztail():
            pltpu.sync_copy(zbuf.at[pl.ds(0, 24)],
                            acc.at[pl.ds(s * RPA + 19 * ZR, 24)])

        @pl.when(s == NS - 1)
        def _ztail_last():
            pltpu.sync_copy(zbuf.at[pl.ds(0, 8)],
                            acc.at[pl.ds(s * RPA + 16 * ZR, 8)])

        wait_idx(0, 0)
        gather(0, 0)
        wait_idx(1, 1)
        gather(1, 1)

        plsc.subcore_barrier()

        def scale(x, g):
            def sg(gi, cc):
                vvec = vv[x][pl.ds(gi * 4, 16)]
                for e in range(4):
                    v = vvec[e]
                    r = gi * 4 + e
                    for t in range(D // 16):
                        gb[g][r, pl.ds(t * 16, 16)] = (
                            gb[g][r, pl.ds(t * 16, 16)] * v)
                return cc

            lax.fori_loop(0, B // 4, sg, 0)

        def step(i, g, x):
            gp2 = (g + 2) % KG
            xp2 = (x + 2) % KI
            xp3 = (x + 3) % KI

            @pl.when(i >= 2)
            def _drain_scatter():
                wait_scatter(xp3, gp2)

            @pl.when(i + 3 <= NCHUNK - 1)
            def _prefetch_idx():
                fetch_idx(i + 3, xp3)

            @pl.when(i + 2 <= NCHUNK - 1)
            def _start_gather():
                wait_idx(i + 2, xp2)
                gather(xp2, gp2)

            wait_gather(x, g)
            scale(x, g)
            scatter(x, g)

        def body(ss, cc):
            for kk in range(SUP):
                step(ss * SUP + kk, kk % KG, kk % KI)
            return cc

        lax.fori_loop(0, NCHUNK // SUP, body, 0)
        for i in range(SUP * (NCHUNK // SUP), NCHUNK):
            step(jnp.int32(i), i % KG, i % KI)

        wait_scatter((NCHUNK - 2) % KI, (NCHUNK - 2) % KG)
        wait_scatter((NCHUNK - 1) % KI, (NCHUNK - 1) % KG)

        plsc.subcore_barrier()

        @pl.when(s < NS - 1)
        def _wb_main():
            pltpu.sync_copy(acc.at[pl.ds(s * RPA, RPA)],
                            out_hbm.at[c, pl.ds(s * RPA, RPA)])

        @pl.when(s == NS - 1)
        def _wb_last():
            pltpu.sync_copy(acc.at[pl.ds((NS - 1) * RPA, RPL)],
                            out_hbm.at[c, pl.ds((NS - 1) * RPA, RPL)])

    return k(xw, rows, cols, vals)


def _add_body(p_ref, o_ref):
    o_ref[...] = p_ref[0] + p_ref[1]


def _combine(partials):
    blk = 1000
    return pl.pallas_call(
        _add_body,
        grid=(N // blk,),
        in_specs=[pl.BlockSpec((NC, blk, D), lambda i: (0, i, 0))],
        out_specs=pl.BlockSpec((blk, D), lambda i: (i, 0)),
        out_shape=jax.ShapeDtypeStruct((N, D), jnp.float32),
    )(partials)


def kernel(x, G_indices, G_values, W, b):
    xw = _xw(x, W, b)
    partials = _spmm_sc(xw, G_indices[0], G_indices[1], G_values)
    return _combine(partials)
